# Initial kernel scaffold; baseline (speedup 1.0000x reference)
import jax
import jax.numpy as jnp
from jax.experimental import pallas as pl


def kernel(x, edge_index):
    raise NotImplementedError("write your pallas kernel here")


# speedup vs baseline: 3.6639x; 3.6639x over previous
import jax
import jax.numpy as jnp
from jax import lax
from jax.experimental import pallas as pl
from jax.experimental.pallas import tpu as pltpu
from jax.experimental.pallas import tpu_sc as plsc

N_NODES = 10000
N_EDGES = 160000
D_FEAT = 256
DH = D_FEAT // 2

NC = 2
NS = 16
LANES = 16

EDGES_PER_TILE = N_EDGES // NS
CHUNK = 80
N_CHUNKS = EDGES_PER_TILE // CHUNK
ROWS_A = 624
ROWS_LAST = N_NODES - 15 * ROWS_A


def _sc_propagate(x2, src, dst, zeros):
    mesh = plsc.VectorSubcoreMesh(
        core_axis_name="c", subcore_axis_name="s", num_cores=NC,
        num_subcores=NS)

    @pl.kernel(
        out_type=jax.ShapeDtypeStruct((NC * N_NODES, DH), jnp.float32),
        mesh=mesh,
        scratch_types=[
            pltpu.VMEM_SHARED((N_NODES, DH), jnp.float32),
            pltpu.VMEM((CHUNK,), jnp.int32),
            pltpu.VMEM((CHUNK,), jnp.int32),
            pltpu.VMEM((CHUNK,), jnp.int32),
            pltpu.VMEM((CHUNK, DH), jnp.float32),
            pltpu.SemaphoreType.DMA,
        ],
    )
    def k(x2_hbm, src_hbm, dst_hbm, zero_hbm, out_hbm,
          accum, src_v, gidx_v, dst_v, rows_v, sem):
        c = lax.axis_index("c")
        s = lax.axis_index("s")

        r0 = pl.multiple_of(s * ROWS_A, 8)

        @pl.when(s < NS - 1)
        def _():
            pltpu.sync_copy(zero_hbm.at[pl.ds(r0, ROWS_A)],
                            accum.at[pl.ds(r0, ROWS_A)])

        @pl.when(s == NS - 1)
        def _():
            pltpu.sync_copy(zero_hbm.at[pl.ds(15 * ROWS_A, ROWS_LAST)],
                            accum.at[pl.ds(15 * ROWS_A, ROWS_LAST)])

        plsc.subcore_barrier()

        base = s * EDGES_PER_TILE
        row_off = c * N_NODES

        def body(i, carry):
            off = base + i * CHUNK
            pltpu.sync_copy(src_hbm.at[pl.ds(off, CHUNK)], src_v)
            pltpu.sync_copy(dst_hbm.at[pl.ds(off, CHUNK)], dst_v)
            for j in range(CHUNK // LANES):
                sl = pl.ds(j * LANES, LANES)
                gidx_v[sl] = src_v[sl] + row_off
            pltpu.async_copy(x2_hbm.at[gidx_v], rows_v, sem).wait()
            pltpu.sync_copy(rows_v, accum.at[dst_v], add=True)
            return carry

        lax.fori_loop(0, N_CHUNKS, body, 0)
        plsc.subcore_barrier()

        o0 = pl.multiple_of(row_off + r0, 8)

        @pl.when(s < NS - 1)
        def _():
            pltpu.sync_copy(accum.at[pl.ds(r0, ROWS_A)],
                            out_hbm.at[pl.ds(o0, ROWS_A)])

        @pl.when(s == NS - 1)
        def _():
            ol = pl.multiple_of(row_off + 15 * ROWS_A, 8)
            pltpu.sync_copy(accum.at[pl.ds(15 * ROWS_A, ROWS_LAST)],
                            out_hbm.at[pl.ds(ol, ROWS_LAST)])

    return k(x2, src, dst, zeros)


def kernel(x, edge_index):
    x2 = jnp.concatenate([x[:, :DH], x[:, DH:]], axis=0)
    src = edge_index[0]
    dst = edge_index[1]
    zeros = jnp.zeros((N_NODES, DH), jnp.float32)
    out2 = _sc_propagate(x2, src, dst, zeros)
    return jnp.concatenate([out2[:N_NODES], out2[N_NODES:]], axis=1)


# speedup vs baseline: 7.7934x; 2.1271x over previous
import jax
import jax.numpy as jnp
from jax import lax
from jax.experimental import pallas as pl
from jax.experimental.pallas import tpu as pltpu
from jax.experimental.pallas import tpu_sc as plsc

N_NODES = 10000
N_EDGES = 160000
D_FEAT = 256
DH = D_FEAT // 2

NC = 2
NS = 16
LANES = 16

CHUNK = 80
CHUNKS_PER_TILE = N_EDGES // NS // CHUNK
N_PAIRS = (CHUNKS_PER_TILE - 1) // 2
ROWS_A = 624
ROWS_LAST = N_NODES - 15 * ROWS_A


def _sc_propagate(x2, src2, dst2, zeros):
    mesh = plsc.VectorSubcoreMesh(
        core_axis_name="c", subcore_axis_name="s", num_cores=NC,
        num_subcores=NS)

    @pl.kernel(
        out_type=jax.ShapeDtypeStruct((NC * N_NODES, DH), jnp.float32),
        mesh=mesh,
        scratch_types=[
            pltpu.VMEM_SHARED((N_NODES, DH), jnp.float32),
            pltpu.VMEM((CHUNKS_PER_TILE * CHUNK,), jnp.int32),
            pltpu.VMEM((CHUNKS_PER_TILE, CHUNK), jnp.int32),
            pltpu.VMEM((2, CHUNK, DH), jnp.float32),
            pltpu.SemaphoreType.DMA,
            pltpu.SemaphoreType.DMA,
            pltpu.SemaphoreType.DMA,
            pltpu.SemaphoreType.DMA,
        ],
    )
    def k(x2_hbm, src2_hbm, dst2_hbm, zero_hbm, out_hbm,
          accum, gidx, didx, rows, gs0, gs1, ss0, ss1):
        c = lax.axis_index("c")
        s = lax.axis_index("s")
        gsem = (gs0, gs1)
        ssem = (ss0, ss1)

        pltpu.sync_copy(src2_hbm.at[c].at[s], gidx)
        pltpu.sync_copy(dst2_hbm.at[s], didx)

        def gslice(i):
            return gidx.at[pl.ds(pl.multiple_of(i * CHUNK, 8), CHUNK)]

        pltpu.async_copy(x2_hbm.at[gslice(0)], rows.at[0], gs0)
        pltpu.async_copy(x2_hbm.at[gslice(1)], rows.at[1], gs1)

        r0 = pl.multiple_of(s * ROWS_A, 8)

        @pl.when(s < NS - 1)
        def _():
            pltpu.sync_copy(zero_hbm.at[pl.ds(r0, ROWS_A)],
                            accum.at[pl.ds(r0, ROWS_A)])

        @pl.when(s == NS - 1)
        def _():
            pltpu.sync_copy(zero_hbm.at[pl.ds(15 * ROWS_A, ROWS_LAST)],
                            accum.at[pl.ds(15 * ROWS_A, ROWS_LAST)])

        plsc.subcore_barrier()

        def do_chunk(i, b, fire_next):
            pltpu.make_async_copy(
                x2_hbm.at[gslice(i)], rows.at[b], gsem[b]).wait()
            pltpu.async_copy(rows.at[b], accum.at[didx.at[i]], ssem[b],
                             add=True)
            pltpu.make_async_copy(
                rows.at[b], accum.at[didx.at[i]], ssem[b]).wait()

            if fire_next:
                @pl.when(i + 2 < CHUNKS_PER_TILE)
                def _():
                    pltpu.async_copy(
                        x2_hbm.at[gslice(i + 2)], rows.at[b], gsem[b])

        def pair(i2, carry):
            do_chunk(2 * i2, 0, True)
            do_chunk(2 * i2 + 1, 1, True)
            return carry

        lax.fori_loop(0, N_PAIRS, pair, 0)
        do_chunk(CHUNKS_PER_TILE - 1, 0, False)

        plsc.subcore_barrier()

        row_off = c * N_NODES
        o0 = pl.multiple_of(row_off + r0, 8)

        @pl.when(s < NS - 1)
        def _():
            pltpu.sync_copy(accum.at[pl.ds(r0, ROWS_A)],
                            out_hbm.at[pl.ds(o0, ROWS_A)])

        @pl.when(s == NS - 1)
        def _():
            ol = pl.multiple_of(row_off + 15 * ROWS_A, 8)
            pltpu.sync_copy(accum.at[pl.ds(15 * ROWS_A, ROWS_LAST)],
                            out_hbm.at[pl.ds(ol, ROWS_LAST)])

    return k(x2, src2, dst2, zeros)


def kernel(x, edge_index):
    x2 = jnp.concatenate([x[:, :DH], x[:, DH:]], axis=0)
    src = edge_index[0]
    dst = edge_index[1]
    src2 = jnp.stack([src, src + N_NODES]).reshape(
        NC, NS, CHUNKS_PER_TILE * CHUNK)
    dst2 = dst.reshape(NS, CHUNKS_PER_TILE, CHUNK)
    zeros = jnp.zeros((N_NODES, DH), jnp.float32)
    out2 = _sc_propagate(x2, src2, dst2, zeros)
    return jnp.concatenate([out2[:N_NODES], out2[N_NODES:]], axis=1)


# speedup vs baseline: 8.9217x; 1.1448x over previous
import jax
import jax.numpy as jnp
from jax import lax
from jax.experimental import pallas as pl
from jax.experimental.pallas import tpu as pltpu
from jax.experimental.pallas import tpu_sc as plsc

N_NODES = 10000
N_EDGES = 160000
D_FEAT = 256
DH = D_FEAT // 2

NC = 2
NS = 16
LANES = 16

CHUNK = 80
EDGES_PER_TILE = N_EDGES // NS
CHUNKS_PER_TILE = EDGES_PER_TILE // CHUNK
N_PAIRS = (CHUNKS_PER_TILE - 1) // 2
ROWS_A = 624
ROWS_LAST = N_NODES - 15 * ROWS_A


def _sc_propagate(x, src1, dst3, zeros):
    mesh = plsc.VectorSubcoreMesh(
        core_axis_name="c", subcore_axis_name="s", num_cores=NC,
        num_subcores=NS)

    @pl.kernel(
        out_type=jax.ShapeDtypeStruct((N_NODES, D_FEAT), jnp.float32),
        mesh=mesh,
        scratch_types=[
            pltpu.VMEM_SHARED((N_NODES, DH), jnp.float32),
            pltpu.VMEM((EDGES_PER_TILE,), jnp.int32),
            pltpu.VMEM((CHUNKS_PER_TILE, CHUNK), jnp.int32),
            pltpu.VMEM((2, CHUNK, DH), jnp.float32),
            pltpu.SemaphoreType.DMA,
            pltpu.SemaphoreType.DMA,
            pltpu.SemaphoreType.DMA,
            pltpu.SemaphoreType.DMA,
        ],
    )
    def k(x_hbm, src_hbm, dst3_hbm, zero_hbm, out_hbm,
          accum, gidx, didx, rows, gs0, gs1, ss0, ss1):
        c = lax.axis_index("c")
        s = lax.axis_index("s")
        gsem = (gs0, gs1)
        ssem = (ss0, ss1)
        col = pl.multiple_of(c * DH, 128)
        xcol = x_hbm.at[:, pl.ds(col, DH)]

        e0 = pl.multiple_of(s * EDGES_PER_TILE, 8)
        pltpu.sync_copy(src_hbm.at[pl.ds(e0, EDGES_PER_TILE)], gidx)
        pltpu.sync_copy(dst3_hbm.at[s], didx)

        def gslice(i):
            return gidx.at[pl.ds(pl.multiple_of(i * CHUNK, 8), CHUNK)]

        pltpu.async_copy(xcol.at[gslice(0)], rows.at[0], gs0)
        pltpu.async_copy(xcol.at[gslice(1)], rows.at[1], gs1)

        r0 = pl.multiple_of(s * ROWS_A, 8)

        @pl.when(s < NS - 1)
        def _():
            pltpu.sync_copy(zero_hbm.at[pl.ds(r0, ROWS_A)],
                            accum.at[pl.ds(r0, ROWS_A)])

        @pl.when(s == NS - 1)
        def _():
            pltpu.sync_copy(zero_hbm.at[pl.ds(15 * ROWS_A, ROWS_LAST)],
                            accum.at[pl.ds(15 * ROWS_A, ROWS_LAST)])

        plsc.subcore_barrier()

        def do_chunk(i, b, fire_next):
            pltpu.make_async_copy(
                xcol.at[gslice(i)], rows.at[b], gsem[b]).wait()
            pltpu.async_copy(rows.at[b], accum.at[didx.at[i]], ssem[b],
                             add=True)
            pltpu.make_async_copy(
                rows.at[b], accum.at[didx.at[i]], ssem[b]).wait()

            if fire_next:
                @pl.when(i + 2 < CHUNKS_PER_TILE)
                def _():
                    pltpu.async_copy(
                        xcol.at[gslice(i + 2)], rows.at[b], gsem[b])

        def pair(i2, carry):
            do_chunk(2 * i2, 0, True)
            do_chunk(2 * i2 + 1, 1, True)
            return carry

        lax.fori_loop(0, N_PAIRS, pair, 0)
        do_chunk(CHUNKS_PER_TILE - 1, 0, False)

        plsc.subcore_barrier()

        @pl.when(s < NS - 1)
        def _():
            pltpu.sync_copy(accum.at[pl.ds(r0, ROWS_A)],
                            out_hbm.at[pl.ds(r0, ROWS_A), pl.ds(col, DH)])

        @pl.when(s == NS - 1)
        def _():
            pltpu.sync_copy(
                accum.at[pl.ds(15 * ROWS_A, ROWS_LAST)],
                out_hbm.at[pl.ds(15 * ROWS_A, ROWS_LAST), pl.ds(col, DH)])

    return k(x, src1, dst3, zeros)


def kernel(x, edge_index):
    src1 = edge_index[0]
    dst3 = edge_index[1].reshape(NS, CHUNKS_PER_TILE, CHUNK)
    zeros = jnp.zeros((N_NODES, DH), jnp.float32)
    return _sc_propagate(x, src1, dst3, zeros)


# speedup vs baseline: 10.4293x; 1.1690x over previous
import jax
import jax.numpy as jnp
from jax import lax
from jax.experimental import pallas as pl
from jax.experimental.pallas import tpu as pltpu
from jax.experimental.pallas import tpu_sc as plsc

N_NODES = 10000
N_EDGES = 160000
D_FEAT = 256
DH = D_FEAT // 2

NC = 2
NS = 16
LANES = 16

CHUNK = 80
EDGES_PER_TILE = N_EDGES // NS
CHUNKS_PER_TILE = EDGES_PER_TILE // CHUNK
N_TRIPLES = (CHUNKS_PER_TILE - 2) // 3
DEPTH = 3
ROWS_A = 624
ROWS_LAST = N_NODES - 15 * ROWS_A


def _sc_propagate(x, src1, dst1, zeros):
    mesh = plsc.VectorSubcoreMesh(
        core_axis_name="c", subcore_axis_name="s", num_cores=NC,
        num_subcores=NS)

    @pl.kernel(
        out_type=jax.ShapeDtypeStruct((N_NODES, D_FEAT), jnp.float32),
        mesh=mesh,
        scratch_types=[
            pltpu.VMEM_SHARED((N_NODES, DH), jnp.float32),
            pltpu.VMEM((EDGES_PER_TILE,), jnp.int32),
            pltpu.VMEM((DEPTH, CHUNK), jnp.int32),
            pltpu.VMEM((DEPTH, CHUNK, DH), jnp.float32),
            [pltpu.SemaphoreType.DMA] * DEPTH,
            [pltpu.SemaphoreType.DMA] * DEPTH,
            [pltpu.SemaphoreType.DMA] * DEPTH,
        ],
    )
    def k(x_hbm, src_hbm, dst_hbm, zero_hbm, out_hbm,
          accum, gidx, didx, rows, gsem, ssem, dsem):
        c = lax.axis_index("c")
        s = lax.axis_index("s")
        col = pl.multiple_of(c * DH, 128)
        xcol = x_hbm.at[:, pl.ds(col, DH)]

        e0 = pl.multiple_of(s * EDGES_PER_TILE, 8)

        def gslice(i):
            return gidx.at[pl.ds(pl.multiple_of(i * CHUNK, 8), CHUNK)]

        def dslice(i):
            return dst_hbm.at[pl.ds(pl.multiple_of(e0 + i * CHUNK, 8), CHUNK)]

        def fire_didx(i, b):
            pltpu.async_copy(dslice(i), didx.at[b], dsem[b])

        def fire_gather(i, b):
            pltpu.async_copy(xcol.at[gslice(i)], rows.at[b], gsem[b])

        pltpu.sync_copy(src_hbm.at[pl.ds(e0, EDGES_PER_TILE)], gidx)
        fire_didx(0, 0)
        fire_didx(1, 1)
        fire_gather(0, 0)
        fire_gather(1, 1)

        r0 = pl.multiple_of(s * ROWS_A, 8)

        @pl.when(s < NS - 1)
        def _():
            pltpu.sync_copy(zero_hbm.at[pl.ds(r0, ROWS_A)],
                            accum.at[pl.ds(r0, ROWS_A)])

        @pl.when(s == NS - 1)
        def _():
            pltpu.sync_copy(zero_hbm.at[pl.ds(15 * ROWS_A, ROWS_LAST)],
                            accum.at[pl.ds(15 * ROWS_A, ROWS_LAST)])

        plsc.subcore_barrier()

        def wait_didx(i, b):
            pltpu.make_async_copy(dslice(i), didx.at[b], dsem[b]).wait()

        def wait_gather(i, b):
            pltpu.make_async_copy(
                xcol.at[gslice(i)], rows.at[b], gsem[b]).wait()

        def fire_scatter(i, b):
            pltpu.async_copy(rows.at[b], accum.at[didx.at[b]], ssem[b],
                             add=True)

        def wait_scatter(i, b):
            pltpu.make_async_copy(
                rows.at[b], accum.at[didx.at[b]], ssem[b]).wait()

        def do_chunk(i, jj, first, fire_next):
            b = jj % DEPTH
            bm1 = (jj - 1) % DEPTH
            wait_didx(i, b)
            wait_gather(i, b)
            fire_scatter(i, b)
            if not first:
                wait_scatter(i - 1, bm1)
            if fire_next:
                fire_didx(i + 2, bm1)
                fire_gather(i + 2, bm1)

        def triple(i2, carry):
            i = 3 * i2

            @pl.when(i2 == 0)
            def _():
                do_chunk(0, 0, True, True)

            @pl.when(i2 > 0)
            def _():
                do_chunk(i, 0, False, True)

            do_chunk(i + 1, 1, False, True)
            do_chunk(i + 2, 2, False, True)
            return carry

        lax.fori_loop(0, N_TRIPLES, triple, 0)
        do_chunk(CHUNKS_PER_TILE - 2, 0, False, False)
        do_chunk(CHUNKS_PER_TILE - 1, 1, False, False)
        wait_scatter(CHUNKS_PER_TILE - 1, 1)

        plsc.subcore_barrier()

        @pl.when(s < NS - 1)
        def _():
            pltpu.sync_copy(accum.at[pl.ds(r0, ROWS_A)],
                            out_hbm.at[pl.ds(r0, ROWS_A), pl.ds(col, DH)])

        @pl.when(s == NS - 1)
        def _():
            pltpu.sync_copy(
                accum.at[pl.ds(15 * ROWS_A, ROWS_LAST)],
                out_hbm.at[pl.ds(15 * ROWS_A, ROWS_LAST), pl.ds(col, DH)])

    return k(x, src1, dst1, zeros)


def kernel(x, edge_index):
    src1 = edge_index[0]
    dst1 = edge_index[1]
    zeros = jnp.zeros((N_NODES, DH), jnp.float32)
    return _sc_propagate(x, src1, dst1, zeros)


# speedup vs baseline: 10.9016x; 1.0453x over previous
import jax
import jax.numpy as jnp
import numpy as np
from jax import lax
from jax.experimental import pallas as pl
from jax.experimental.pallas import tpu as pltpu
from jax.experimental.pallas import tpu_sc as plsc

N_NODES = 10000
N_EDGES = 160000
D_FEAT = 256
DH = D_FEAT // 2

NC = 2
NS = 16
LANES = 16

CHUNK = 80
EDGES_PER_TILE = N_EDGES // NS
CHUNKS_PER_TILE = EDGES_PER_TILE // CHUNK
N_TRIPLES = (CHUNKS_PER_TILE - 2) // 3
DEPTH = 3
ROWS_A = 624
ROWS_LAST = N_NODES - 15 * ROWS_A


def _sc_propagate(x, edge1, zeros):
    mesh = plsc.VectorSubcoreMesh(
        core_axis_name="c", subcore_axis_name="s", num_cores=NC,
        num_subcores=NS)

    @pl.kernel(
        out_type=jax.ShapeDtypeStruct((N_NODES, D_FEAT), jnp.float32),
        mesh=mesh,
        scratch_types=[
            pltpu.VMEM_SHARED((N_NODES, DH), jnp.float32),
            pltpu.VMEM((EDGES_PER_TILE,), jnp.int32),
            pltpu.VMEM((DEPTH, CHUNK), jnp.int32),
            pltpu.VMEM((DEPTH, CHUNK, DH), jnp.float32),
            [pltpu.SemaphoreType.DMA] * DEPTH,
            [pltpu.SemaphoreType.DMA] * DEPTH,
            [pltpu.SemaphoreType.DMA] * DEPTH,
        ],
    )
    def k(x_hbm, edge_hbm, zero_hbm, out_hbm,
          accum, gidx, didx, rows, gsem, ssem, dsem):
        c = lax.axis_index("c")
        s = lax.axis_index("s")
        col = pl.multiple_of(c * DH, 128)
        xcol = x_hbm.at[:, pl.ds(col, DH)]

        e0 = pl.multiple_of(s * EDGES_PER_TILE, 8)

        def gslice(i):
            return gidx.at[pl.ds(pl.multiple_of(i * CHUNK, 8), CHUNK)]

        def dslice(i):
            return edge_hbm.at[
                pl.ds(pl.multiple_of(N_EDGES + e0 + i * CHUNK, 8), CHUNK)]

        def fire_didx(i, b):
            pltpu.async_copy(dslice(i), didx.at[b], dsem[b])

        def fire_gather(i, b):
            pltpu.async_copy(xcol.at[gslice(i)], rows.at[b], gsem[b])

        pltpu.sync_copy(edge_hbm.at[pl.ds(e0, EDGES_PER_TILE)], gidx)
        fire_didx(0, 0)
        fire_didx(1, 1)
        fire_gather(0, 0)
        fire_gather(1, 1)

        r0 = pl.multiple_of(s * ROWS_A, 8)

        @pl.when(s < NS - 1)
        def _():
            pltpu.sync_copy(zero_hbm.at[pl.ds(r0, ROWS_A)],
                            accum.at[pl.ds(r0, ROWS_A)])

        @pl.when(s == NS - 1)
        def _():
            pltpu.sync_copy(zero_hbm.at[pl.ds(15 * ROWS_A, ROWS_LAST)],
                            accum.at[pl.ds(15 * ROWS_A, ROWS_LAST)])

        plsc.subcore_barrier()

        def wait_didx(i, b):
            pltpu.make_async_copy(dslice(i), didx.at[b], dsem[b]).wait()

        def wait_gather(i, b):
            pltpu.make_async_copy(
                xcol.at[gslice(i)], rows.at[b], gsem[b]).wait()

        def fire_scatter(i, b):
            pltpu.async_copy(rows.at[b], accum.at[didx.at[b]], ssem[b],
                             add=True)

        def wait_scatter(i, b):
            pltpu.make_async_copy(
                rows.at[b], accum.at[didx.at[b]], ssem[b]).wait()

        def do_chunk(i, jj, first, fire_next):
            b = jj % DEPTH
            bm1 = (jj - 1) % DEPTH
            wait_didx(i, b)
            wait_gather(i, b)
            fire_scatter(i, b)
            if not first:
                wait_scatter(i - 1, bm1)
            if fire_next:
                fire_didx(i + 2, bm1)
                fire_gather(i + 2, bm1)

        def triple(i2, carry):
            i = 3 * i2

            @pl.when(i2 == 0)
            def _():
                do_chunk(0, 0, True, True)

            @pl.when(i2 > 0)
            def _():
                do_chunk(i, 0, False, True)

            do_chunk(i + 1, 1, False, True)
            do_chunk(i + 2, 2, False, True)
            return carry

        lax.fori_loop(0, N_TRIPLES, triple, 0)
        do_chunk(CHUNKS_PER_TILE - 2, 0, False, False)
        do_chunk(CHUNKS_PER_TILE - 1, 1, False, False)
        wait_scatter(CHUNKS_PER_TILE - 1, 1)

        plsc.subcore_barrier()

        @pl.when(s < NS - 1)
        def _():
            pltpu.sync_copy(accum.at[pl.ds(r0, ROWS_A)],
                            out_hbm.at[pl.ds(r0, ROWS_A), pl.ds(col, DH)])

        @pl.when(s == NS - 1)
        def _():
            pltpu.sync_copy(
                accum.at[pl.ds(15 * ROWS_A, ROWS_LAST)],
                out_hbm.at[pl.ds(15 * ROWS_A, ROWS_LAST), pl.ds(col, DH)])

    return k(x, edge1, zeros)


_ZEROS = np.zeros((N_NODES, DH), np.float32)


def kernel(x, edge_index):
    edge1 = edge_index.reshape(2 * N_EDGES)
    return _sc_propagate(x, edge1, jnp.asarray(_ZEROS))


# speedup vs baseline: 11.3064x; 1.0371x over previous
import jax
import jax.numpy as jnp
from jax import lax
from jax.experimental import pallas as pl
from jax.experimental.pallas import tpu as pltpu
from jax.experimental.pallas import tpu_sc as plsc

N_NODES = 10000
N_EDGES = 160000
D_FEAT = 256
DH = D_FEAT // 2

NC = 2
NS = 16
LANES = 16

CHUNK = 80
EDGES_PER_TILE = N_EDGES // NS
CHUNKS_PER_TILE = EDGES_PER_TILE // CHUNK
N_TRIPLES = (CHUNKS_PER_TILE - 2) // 3
DEPTH = 3
ROWS_A = 624
ROWS_LAST = N_NODES - 15 * ROWS_A


def _sc_propagate(x, edge1):
    mesh = plsc.VectorSubcoreMesh(
        core_axis_name="c", subcore_axis_name="s", num_cores=NC,
        num_subcores=NS)

    @pl.kernel(
        out_type=jax.ShapeDtypeStruct((N_NODES, D_FEAT), jnp.float32),
        mesh=mesh,
        scratch_types=[
            pltpu.VMEM_SHARED((N_NODES, DH), jnp.float32),
            pltpu.VMEM((EDGES_PER_TILE,), jnp.int32),
            pltpu.VMEM((DEPTH, CHUNK), jnp.int32),
            pltpu.VMEM((DEPTH, CHUNK, DH), jnp.float32),
            [pltpu.SemaphoreType.DMA] * DEPTH,
            [pltpu.SemaphoreType.DMA] * DEPTH,
            [pltpu.SemaphoreType.DMA] * DEPTH,
        ],
    )
    def k(x_hbm, edge_hbm, out_hbm,
          accum, gidx, didx, rows, gsem, ssem, dsem):
        c = lax.axis_index("c")
        s = lax.axis_index("s")
        col = pl.multiple_of(c * DH, 128)
        xcol = x_hbm.at[:, pl.ds(col, DH)]

        e0 = pl.multiple_of(s * EDGES_PER_TILE, 8)

        def gslice(i):
            return gidx.at[pl.ds(pl.multiple_of(i * CHUNK, 8), CHUNK)]

        def dslice(i):
            return edge_hbm.at[
                pl.ds(pl.multiple_of(N_EDGES + e0 + i * CHUNK, 8), CHUNK)]

        def fire_didx(i, b):
            pltpu.async_copy(dslice(i), didx.at[b], dsem[b])

        def fire_gather(i, b):
            pltpu.async_copy(xcol.at[gslice(i)], rows.at[b], gsem[b])

        pltpu.sync_copy(edge_hbm.at[pl.ds(e0, EDGES_PER_TILE)], gidx)
        fire_didx(0, 0)
        fire_didx(1, 1)
        fire_gather(0, 0)
        fire_gather(1, 1)

        zvec = jnp.zeros((LANES,), jnp.float32)

        def zrow(r, carry):
            for j in range(DH // LANES):
                rows[2, r, pl.ds(j * LANES, LANES)] = zvec
            return carry

        lax.fori_loop(0, CHUNK, zrow, 0)

        r0 = pl.multiple_of(s * ROWS_A, 8)

        @pl.when(s < NS - 1)
        def _():
            for kk in range(ROWS_A // CHUNK):
                pltpu.sync_copy(rows.at[2],
                                accum.at[pl.ds(r0 + kk * CHUNK, CHUNK)])
            rem = ROWS_A - (ROWS_A // CHUNK) * CHUNK
            pltpu.sync_copy(
                rows.at[2].at[pl.ds(0, rem)],
                accum.at[pl.ds(r0 + ROWS_A - rem, rem)])

        @pl.when(s == NS - 1)
        def _():
            for kk in range(ROWS_LAST // CHUNK):
                pltpu.sync_copy(
                    rows.at[2],
                    accum.at[pl.ds(15 * ROWS_A + kk * CHUNK, CHUNK)])

        plsc.subcore_barrier()

        def wait_didx(i, b):
            pltpu.make_async_copy(dslice(i), didx.at[b], dsem[b]).wait()

        def wait_gather(i, b):
            pltpu.make_async_copy(
                xcol.at[gslice(i)], rows.at[b], gsem[b]).wait()

        def fire_scatter(i, b):
            pltpu.async_copy(rows.at[b], accum.at[didx.at[b]], ssem[b],
                             add=True)

        def wait_scatter(i, b):
            pltpu.make_async_copy(
                rows.at[b], accum.at[didx.at[b]], ssem[b]).wait()

        def do_chunk(i, jj, first, fire_next):
            b = jj % DEPTH
            bm1 = (jj - 1) % DEPTH
            wait_didx(i, b)
            wait_gather(i, b)
            fire_scatter(i, b)
            if not first:
                wait_scatter(i - 1, bm1)
            if fire_next:
                fire_didx(i + 2, bm1)
                fire_gather(i + 2, bm1)

        def triple(i2, carry):
            i = 3 * i2

            @pl.when(i2 == 0)
            def _():
                do_chunk(0, 0, True, True)

            @pl.when(i2 > 0)
            def _():
                do_chunk(i, 0, False, True)

            do_chunk(i + 1, 1, False, True)
            do_chunk(i + 2, 2, False, True)
            return carry

        lax.fori_loop(0, N_TRIPLES, triple, 0)
        do_chunk(CHUNKS_PER_TILE - 2, 0, False, False)
        do_chunk(CHUNKS_PER_TILE - 1, 1, False, False)
        wait_scatter(CHUNKS_PER_TILE - 1, 1)

        plsc.subcore_barrier()

        @pl.when(s < NS - 1)
        def _():
            pltpu.sync_copy(accum.at[pl.ds(r0, ROWS_A)],
                            out_hbm.at[pl.ds(r0, ROWS_A), pl.ds(col, DH)])

        @pl.when(s == NS - 1)
        def _():
            pltpu.sync_copy(
                accum.at[pl.ds(15 * ROWS_A, ROWS_LAST)],
                out_hbm.at[pl.ds(15 * ROWS_A, ROWS_LAST), pl.ds(col, DH)])

    return k(x, edge1)


def kernel(x, edge_index):
    edge1 = edge_index.reshape(2 * N_EDGES)
    return _sc_propagate(x, edge1)
